# baseline (speedup 1.0000x reference)
import jax
import jax.numpy as jnp
from jax.experimental import pallas as pl
from jax.experimental.pallas import tpu as pltpu

_CAP = 16384
_D = 3 * 32 * 32


def _scatter_body(idx_ref, bx_any, by_any, bt_any, x_ref, y_ref, t_ref,
                  obx_ref, oby_ref, obt_ref):
    del idx_ref, bx_any, by_any, bt_any
    obx_ref[...] = x_ref[...]
    oby_ref[...] = y_ref[...]
    obt_ref[...] = t_ref[...]


def kernel(bx, by, bt, x, y, t, idx):
    B = x.shape[0]
    x3 = x.reshape(B, 1, _D)
    bx3 = bx.reshape(_CAP, 1, _D)
    y3 = y.reshape(B, 1, 1)
    t3 = t.reshape(B, 1, 1)
    by3 = by.reshape(_CAP, 1, 1)
    bt3 = bt.reshape(_CAP, 1, 1)

    grid_spec = pltpu.PrefetchScalarGridSpec(
        num_scalar_prefetch=1,
        grid=(B,),
        in_specs=[
            pl.BlockSpec(memory_space=pl.ANY),
            pl.BlockSpec(memory_space=pl.ANY),
            pl.BlockSpec(memory_space=pl.ANY),
            pl.BlockSpec((1, 1, _D), lambda i, idx_ref: (i, 0, 0)),
            pl.BlockSpec((1, 1, 1), lambda i, idx_ref: (i, 0, 0)),
            pl.BlockSpec((1, 1, 1), lambda i, idx_ref: (i, 0, 0)),
        ],
        out_specs=[
            pl.BlockSpec((1, 1, _D), lambda i, idx_ref: (idx_ref[i], 0, 0)),
            pl.BlockSpec((1, 1, 1), lambda i, idx_ref: (idx_ref[i], 0, 0)),
            pl.BlockSpec((1, 1, 1), lambda i, idx_ref: (idx_ref[i], 0, 0)),
        ],
    )
    obx, oby, obt = pl.pallas_call(
        _scatter_body,
        grid_spec=grid_spec,
        out_shape=[
            jax.ShapeDtypeStruct((_CAP, 1, _D), jnp.float32),
            jax.ShapeDtypeStruct((_CAP, 1, 1), jnp.int32),
            jax.ShapeDtypeStruct((_CAP, 1, 1), jnp.int32),
        ],
        input_output_aliases={1: 0, 2: 1, 3: 2},
    )(idx, bx3, by3, bt3, x3, y3, t3)
    return (obx.reshape(_CAP, 3, 32, 32), oby.reshape(_CAP), obt.reshape(_CAP))
